# baseline (speedup 1.0000x reference)
import functools

import jax
import jax.numpy as jnp
from jax import lax
from jax.experimental import pallas as pl
from jax.experimental.pallas import tpu as pltpu
from jax.experimental.pallas import tpu_sc as plsc

N = 10000
E = 320000
C = 128
CA = 144
NUM_CORES = 2
NUM_SUBCORES = 16
NW = NUM_CORES * NUM_SUBCORES
EW = E // NW
K = 128
NB = 80
EP = NW * NB * K
RPT = 632
N_ACC = RPT * NUM_SUBCORES


def _sc_aggregate(x_aug, row3, col3, zeros_nca):
  mesh = plsc.VectorSubcoreMesh(core_axis_name="c", subcore_axis_name="s")

  @functools.partial(
      pl.kernel,
      out_type=jax.ShapeDtypeStruct((NUM_CORES, N_ACC, CA), jnp.float32),
      mesh=mesh,
      compiler_params=pltpu.CompilerParams(use_tc_tiling_on_sc=False),
      scratch_types=[
          pltpu.VMEM((4, K), jnp.int32),
          pltpu.VMEM((4, K), jnp.int32),
          pltpu.VMEM((2, K, CA), jnp.float32),
          pltpu.VMEM_SHARED((N_ACC, CA), jnp.float32),
          pltpu.SemaphoreType.DMA,
          pltpu.SemaphoreType.DMA,
          pltpu.SemaphoreType.DMA,
      ],
  )
  def body(x_hbm, row_hbm, col_hbm, zero_hbm, out_hbm,
           row_v, col_v, gbuf, acc_sh, sem_i, sem_g, sem_s):
    cid = lax.axis_index("c")
    sid = lax.axis_index("s")
    wid = (1 - cid) * NUM_SUBCORES + sid

    pltpu.sync_copy(zero_hbm.at[pl.ds(sid * RPT, RPT)],
                    acc_sh.at[pl.ds(sid * RPT, RPT)])
    pltpu.async_copy(row_hbm.at[wid, 0], row_v.at[0], sem_i)
    pltpu.async_copy(col_hbm.at[wid, 0], col_v.at[0], sem_i)
    pltpu.async_copy(row_hbm.at[wid, 1], row_v.at[1], sem_i)
    pltpu.async_copy(col_hbm.at[wid, 1], col_v.at[1], sem_i)
    plsc.subcore_barrier()
    pltpu.make_async_copy(row_hbm.at[wid, 0], row_v.at[0], sem_i).wait()
    pltpu.make_async_copy(col_hbm.at[wid, 0], col_v.at[0], sem_i).wait()
    pltpu.async_copy(x_hbm.at[col_v.at[0]], gbuf.at[0], sem_g)

    def step(j, _):
      s2 = lax.rem(j, 2)
      ns2 = lax.rem(j + 1, 2)
      s4 = lax.rem(j, 4)

      @pl.when(j >= 1)
      def _():
        pltpu.make_async_copy(
            gbuf.at[ns2], acc_sh.at[row_v.at[lax.rem(j + 3, 4)]],
            sem_s).wait()

      @pl.when(j + 1 < NB)
      def _():
        pltpu.make_async_copy(row_hbm.at[wid, j + 1],
                              row_v.at[lax.rem(j + 1, 4)], sem_i).wait()
        pltpu.make_async_copy(col_hbm.at[wid, j + 1],
                              col_v.at[lax.rem(j + 1, 4)], sem_i).wait()
        pltpu.async_copy(x_hbm.at[col_v.at[lax.rem(j + 1, 4)]],
                         gbuf.at[ns2], sem_g)

      @pl.when(j + 2 < NB)
      def _():
        pltpu.async_copy(row_hbm.at[wid, j + 2],
                         row_v.at[lax.rem(j + 2, 4)], sem_i)
        pltpu.async_copy(col_hbm.at[wid, j + 2],
                         col_v.at[lax.rem(j + 2, 4)], sem_i)

      pltpu.make_async_copy(x_hbm.at[col_v.at[s4]], gbuf.at[s2],
                            sem_g).wait()
      pltpu.async_copy(gbuf.at[s2], acc_sh.at[row_v.at[s4]], sem_s,
                       add=True)
      return 0

    lax.fori_loop(0, NB, step, 0)
    pltpu.make_async_copy(gbuf.at[(NB - 1) % 2],
                          acc_sh.at[row_v.at[(NB - 1) % 4]], sem_s).wait()
    plsc.subcore_barrier()

    pltpu.sync_copy(acc_sh.at[pl.ds(sid * RPT, RPT)],
                    out_hbm.at[cid, pl.ds(sid * RPT, RPT)])

  return body(x_aug, row3, col3, zeros_nca)


def _combine(agg0, agg1, deg0, deg1, w_t, b_row):
  BM = 1000

  def body(a0, a1, d0, d1, wt, bb, o):
    s = a0[...] + a1[...]
    d = d0[...] + d1[...]
    o[...] = jnp.dot(s / d, wt[...],
                     preferred_element_type=jnp.float32) + bb[...]

  return pl.pallas_call(
      body,
      grid=(N // BM,),
      in_specs=[
          pl.BlockSpec((BM, C), lambda i: (i, 0)),
          pl.BlockSpec((BM, C), lambda i: (i, 0)),
          pl.BlockSpec((BM, 1), lambda i: (i, 0)),
          pl.BlockSpec((BM, 1), lambda i: (i, 0)),
          pl.BlockSpec((C, C), lambda i: (0, 0)),
          pl.BlockSpec((1, C), lambda i: (0, 0)),
      ],
      out_specs=pl.BlockSpec((BM, C), lambda i: (i, 0)),
      out_shape=jax.ShapeDtypeStruct((N, C), jnp.float32),
  )(agg0, agg1, deg0, deg1, w_t, b_row)


def kernel(node_feats, edge_index, W, b):
  x_aug = jnp.zeros((N, CA), jnp.float32)
  x_aug = x_aug.at[:, :C].set(node_feats)
  x_aug = x_aug.at[:, C].set(1.0)
  pad = EP - E
  pad_rows = N + jnp.arange(pad, dtype=jnp.int32) % (N_ACC - N)
  row3 = jnp.concatenate([edge_index[0], pad_rows]).reshape(NW, NB, K)
  col3 = jnp.concatenate(
      [edge_index[1], jnp.zeros((pad,), jnp.int32)]).reshape(NW, NB, K)
  zeros_nca = jnp.zeros((N_ACC, CA), jnp.float32)

  part = _sc_aggregate(x_aug, row3, col3, zeros_nca)

  agg0 = part[0, :N, :C]
  agg1 = part[1, :N, :C]
  deg0 = part[0, :N, C:C + 1]
  deg1 = part[1, :N, C:C + 1]
  return _combine(agg0, agg1, deg0, deg1, W.T, b.reshape(1, C))


# speedup vs baseline: 2.1778x; 2.1778x over previous
import functools

import jax
import jax.numpy as jnp
from jax import lax
from jax.experimental import pallas as pl
from jax.experimental.pallas import tpu as pltpu
from jax.experimental.pallas import tpu_sc as plsc

N = 10000
E = 320000
C = 128
CA = 144
NUM_CORES = 2
NUM_SUBCORES = 16
NW = NUM_CORES * NUM_SUBCORES
EW = E // NW
K = 128
NB = 80
EP = NW * NB * K
RPT = 632
N_ACC = RPT * NUM_SUBCORES


def _sc_aggregate(x_aug, row3, col3, zeros_nca):
  mesh = plsc.VectorSubcoreMesh(core_axis_name="c", subcore_axis_name="s")

  @functools.partial(
      pl.kernel,
      out_type=jax.ShapeDtypeStruct((NUM_CORES, N_ACC, CA), jnp.float32),
      mesh=mesh,
      compiler_params=pltpu.CompilerParams(use_tc_tiling_on_sc=False),
      scratch_types=[
          pltpu.VMEM((4, K), jnp.int32),
          pltpu.VMEM((4, K), jnp.int32),
          pltpu.VMEM((2, K, CA), jnp.float32),
          pltpu.VMEM_SHARED((N_ACC, CA), jnp.float32),
          pltpu.SemaphoreType.DMA,
          pltpu.SemaphoreType.DMA,
          pltpu.SemaphoreType.DMA,
      ],
  )
  def body(x_hbm, row_hbm, col_hbm, zero_hbm, out_hbm,
           row_v, col_v, gbuf, acc_sh, sem_i, sem_g, sem_s):
    cid = lax.axis_index("c")
    sid = lax.axis_index("s")
    wid = cid * NUM_SUBCORES + sid

    pltpu.sync_copy(zero_hbm.at[pl.ds(sid * RPT, RPT)],
                    acc_sh.at[pl.ds(sid * RPT, RPT)])
    pltpu.async_copy(row_hbm.at[wid, 0], row_v.at[0], sem_i)
    pltpu.async_copy(col_hbm.at[wid, 0], col_v.at[0], sem_i)
    pltpu.async_copy(row_hbm.at[wid, 1], row_v.at[1], sem_i)
    pltpu.async_copy(col_hbm.at[wid, 1], col_v.at[1], sem_i)
    plsc.subcore_barrier()
    pltpu.make_async_copy(row_hbm.at[wid, 0], row_v.at[0], sem_i).wait()
    pltpu.make_async_copy(col_hbm.at[wid, 0], col_v.at[0], sem_i).wait()
    pltpu.async_copy(x_hbm.at[col_v.at[0]], gbuf.at[0], sem_g)

    def step(j, _):
      s2 = lax.rem(j, 2)
      ns2 = lax.rem(j + 1, 2)
      s4 = lax.rem(j, 4)

      @pl.when(j >= 1)
      def _():
        pltpu.make_async_copy(
            gbuf.at[ns2], acc_sh.at[row_v.at[lax.rem(j + 3, 4)]],
            sem_s).wait()

      @pl.when(j + 1 < NB)
      def _():
        pltpu.make_async_copy(row_hbm.at[wid, j + 1],
                              row_v.at[lax.rem(j + 1, 4)], sem_i).wait()
        pltpu.make_async_copy(col_hbm.at[wid, j + 1],
                              col_v.at[lax.rem(j + 1, 4)], sem_i).wait()
        pltpu.async_copy(x_hbm.at[col_v.at[lax.rem(j + 1, 4)]],
                         gbuf.at[ns2], sem_g)

      @pl.when(j + 2 < NB)
      def _():
        pltpu.async_copy(row_hbm.at[wid, j + 2],
                         row_v.at[lax.rem(j + 2, 4)], sem_i)
        pltpu.async_copy(col_hbm.at[wid, j + 2],
                         col_v.at[lax.rem(j + 2, 4)], sem_i)

      pltpu.make_async_copy(x_hbm.at[col_v.at[s4]], gbuf.at[s2],
                            sem_g).wait()
      pltpu.async_copy(gbuf.at[s2], acc_sh.at[row_v.at[s4]], sem_s,
                       add=True)
      return 0

    lax.fori_loop(0, NB, step, 0)
    pltpu.make_async_copy(gbuf.at[(NB - 1) % 2],
                          acc_sh.at[row_v.at[(NB - 1) % 4]], sem_s).wait()
    plsc.subcore_barrier()

    pltpu.sync_copy(acc_sh.at[pl.ds(sid * RPT, RPT)],
                    out_hbm.at[cid, pl.ds(sid * RPT, RPT)])

  return body(x_aug, row3, col3, zeros_nca)


def _combine(agg0, agg1, deg0, deg1, w_t, b_row):
  BM = 1000

  def body(a0, a1, d0, d1, wt, bb, o):
    s = a0[...] + a1[...]
    d = d0[...] + d1[...]
    o[...] = jnp.dot(s / d, wt[...],
                     preferred_element_type=jnp.float32) + bb[...]

  return pl.pallas_call(
      body,
      grid=(N // BM,),
      in_specs=[
          pl.BlockSpec((BM, C), lambda i: (i, 0)),
          pl.BlockSpec((BM, C), lambda i: (i, 0)),
          pl.BlockSpec((BM, 1), lambda i: (i, 0)),
          pl.BlockSpec((BM, 1), lambda i: (i, 0)),
          pl.BlockSpec((C, C), lambda i: (0, 0)),
          pl.BlockSpec((1, C), lambda i: (0, 0)),
      ],
      out_specs=pl.BlockSpec((BM, C), lambda i: (i, 0)),
      out_shape=jax.ShapeDtypeStruct((N, C), jnp.float32),
  )(agg0, agg1, deg0, deg1, w_t, b_row)


def kernel(node_feats, edge_index, W, b):
  x_aug = jnp.zeros((N, CA), jnp.float32)
  x_aug = x_aug.at[:, :C].set(node_feats)
  x_aug = x_aug.at[:, C].set(1.0)
  pad = EP - E
  ar = jnp.arange(pad, dtype=jnp.int32)
  pad_rows = N + ar % (N_ACC - N)
  pad_cols = ar % N
  row3 = jnp.concatenate([edge_index[0], pad_rows]).reshape(NW, NB, K)
  col3 = jnp.concatenate([edge_index[1], pad_cols]).reshape(NW, NB, K)
  zeros_nca = jnp.zeros((N_ACC, CA), jnp.float32)

  part = _sc_aggregate(x_aug, row3, col3, zeros_nca)

  agg0 = part[0, :N, :C]
  agg1 = part[1, :N, :C]
  deg0 = part[0, :N, C:C + 1]
  deg1 = part[1, :N, C:C + 1]
  return _combine(agg0, agg1, deg0, deg1, W.T, b.reshape(1, C))


# speedup vs baseline: 2.6255x; 1.2056x over previous
import functools

import jax
import jax.numpy as jnp
from jax import lax
from jax.experimental import pallas as pl
from jax.experimental.pallas import tpu as pltpu
from jax.experimental.pallas import tpu_sc as plsc

N = 10000
E = 320000
C = 128
CA = 144
CD = 16
NUM_CORES = 2
NUM_SUBCORES = 16
NW = NUM_CORES * NUM_SUBCORES
EW = E // NW
K = 128
NB = 80
PADW = NB * K - EW
RPT = 632
N_ACC = RPT * NUM_SUBCORES


def _sc_aggregate(x_aug, row3, col3, zeros_nca):
  mesh = plsc.VectorSubcoreMesh(core_axis_name="c", subcore_axis_name="s")

  @functools.partial(
      pl.kernel,
      out_type=(jax.ShapeDtypeStruct((NUM_CORES, N_ACC, C), jnp.float32),
                jax.ShapeDtypeStruct((NUM_CORES, N_ACC, CD), jnp.float32)),
      mesh=mesh,
      compiler_params=pltpu.CompilerParams(use_tc_tiling_on_sc=False),
      scratch_types=[
          pltpu.VMEM((4, K), jnp.int32),
          pltpu.VMEM((4, K), jnp.int32),
          pltpu.VMEM((2, K, CA), jnp.float32),
          pltpu.VMEM_SHARED((N_ACC, CA), jnp.float32),
          pltpu.SemaphoreType.DMA,
          pltpu.SemaphoreType.DMA,
          pltpu.SemaphoreType.DMA,
      ],
  )
  def body(x_hbm, row_hbm, col_hbm, zero_hbm, agg_hbm, deg_hbm,
           row_v, col_v, gbuf, acc_sh, sem_i, sem_g, sem_s):
    cid = lax.axis_index("c")
    sid = lax.axis_index("s")
    wid = cid * NUM_SUBCORES + sid

    pltpu.sync_copy(zero_hbm.at[pl.ds(sid * RPT, RPT)],
                    acc_sh.at[pl.ds(sid * RPT, RPT)])
    pltpu.async_copy(row_hbm.at[wid, 0], row_v.at[0], sem_i)
    pltpu.async_copy(col_hbm.at[wid, 0], col_v.at[0], sem_i)
    pltpu.async_copy(row_hbm.at[wid, 1], row_v.at[1], sem_i)
    pltpu.async_copy(col_hbm.at[wid, 1], col_v.at[1], sem_i)
    plsc.subcore_barrier()
    pltpu.make_async_copy(row_hbm.at[wid, 0], row_v.at[0], sem_i).wait()
    pltpu.make_async_copy(col_hbm.at[wid, 0], col_v.at[0], sem_i).wait()
    pltpu.async_copy(x_hbm.at[col_v.at[0]], gbuf.at[0], sem_g)

    def step(j, _):
      s2 = lax.rem(j, 2)
      ns2 = lax.rem(j + 1, 2)
      s4 = lax.rem(j, 4)

      @pl.when(j >= 1)
      def _():
        pltpu.make_async_copy(
            gbuf.at[ns2], acc_sh.at[row_v.at[lax.rem(j + 3, 4)]],
            sem_s).wait()

      @pl.when(j + 1 < NB)
      def _():
        pltpu.make_async_copy(row_hbm.at[wid, j + 1],
                              row_v.at[lax.rem(j + 1, 4)], sem_i).wait()
        pltpu.make_async_copy(col_hbm.at[wid, j + 1],
                              col_v.at[lax.rem(j + 1, 4)], sem_i).wait()
        pltpu.async_copy(x_hbm.at[col_v.at[lax.rem(j + 1, 4)]],
                         gbuf.at[ns2], sem_g)

      @pl.when(j + 2 < NB)
      def _():
        pltpu.async_copy(row_hbm.at[wid, j + 2],
                         row_v.at[lax.rem(j + 2, 4)], sem_i)
        pltpu.async_copy(col_hbm.at[wid, j + 2],
                         col_v.at[lax.rem(j + 2, 4)], sem_i)

      pltpu.make_async_copy(x_hbm.at[col_v.at[s4]], gbuf.at[s2],
                            sem_g).wait()
      pltpu.async_copy(gbuf.at[s2], acc_sh.at[row_v.at[s4]], sem_s,
                       add=True)
      return 0

    lax.fori_loop(0, NB, step, 0)
    pltpu.make_async_copy(gbuf.at[(NB - 1) % 2],
                          acc_sh.at[row_v.at[(NB - 1) % 4]], sem_s).wait()
    plsc.subcore_barrier()

    rows = pl.ds(sid * RPT, RPT)
    pltpu.sync_copy(acc_sh.at[rows, pl.ds(0, C)], agg_hbm.at[cid, rows])
    pltpu.sync_copy(acc_sh.at[rows, pl.ds(C, CD)], deg_hbm.at[cid, rows])

  return body(x_aug, row3, col3, zeros_nca)


def _combine(agg, deg, w_t, b_row):
  BM = 1000

  def body(a0, a1, d0, d1, wt, bb, o):
    s = a0[0] + a1[0]
    d = d0[0, :, 0:1] + d1[0, :, 0:1]
    o[...] = jnp.dot(s / d, wt[...],
                     preferred_element_type=jnp.float32) + bb[...]

  return pl.pallas_call(
      body,
      grid=(N // BM,),
      in_specs=[
          pl.BlockSpec((1, BM, C), lambda i: (0, i, 0)),
          pl.BlockSpec((1, BM, C), lambda i: (1, i, 0)),
          pl.BlockSpec((1, BM, CD), lambda i: (0, i, 0)),
          pl.BlockSpec((1, BM, CD), lambda i: (1, i, 0)),
          pl.BlockSpec((C, C), lambda i: (0, 0)),
          pl.BlockSpec((1, C), lambda i: (0, 0)),
      ],
      out_specs=pl.BlockSpec((BM, C), lambda i: (i, 0)),
      out_shape=jax.ShapeDtypeStruct((N, C), jnp.float32),
  )(agg, agg, deg, deg, w_t, b_row)


def kernel(node_feats, edge_index, W, b):
  ones_pad = jnp.concatenate(
      [jnp.ones((N, 1), jnp.float32), jnp.zeros((N, CA - C - 1), jnp.float32)],
      axis=1)
  x_aug = jnp.concatenate([node_feats, ones_pad], axis=1)
  ar = jnp.arange(NW * PADW, dtype=jnp.int32).reshape(NW, PADW)
  pad_rows = N + ar % (N_ACC - N)
  pad_cols = (ar * 37) % N
  row3 = jnp.concatenate(
      [edge_index[0].reshape(NW, EW), pad_rows], axis=1).reshape(NW, NB, K)
  col3 = jnp.concatenate(
      [edge_index[1].reshape(NW, EW), pad_cols], axis=1).reshape(NW, NB, K)
  zeros_nca = jnp.zeros((N_ACC, CA), jnp.float32)

  agg, deg = _sc_aggregate(x_aug, row3, col3, zeros_nca)
  return _combine(agg, deg, W.T, b.reshape(1, C))


# speedup vs baseline: 3.3421x; 1.2729x over previous
import functools

import jax
import jax.numpy as jnp
from jax import lax
from jax.experimental import pallas as pl
from jax.experimental.pallas import tpu as pltpu
from jax.experimental.pallas import tpu_sc as plsc

N = 10000
E = 320000
C = 128
CD = 16
NUM_CORES = 2
NUM_SUBCORES = 16
NW = NUM_CORES * NUM_SUBCORES
EW = E // NW
K = 128
NBF = EW // K
REM = EW - NBF * K
RPT = 632
N_ACC = RPT * NUM_SUBCORES


def _sc_aggregate(node_feats, edge_hbm, zeros_f, zeros_d, ones_kd):
  mesh = plsc.VectorSubcoreMesh(core_axis_name="c", subcore_axis_name="s")

  @functools.partial(
      pl.kernel,
      out_type=(jax.ShapeDtypeStruct((NUM_CORES, N_ACC, C), jnp.float32),
                jax.ShapeDtypeStruct((NUM_CORES, N_ACC, CD), jnp.float32)),
      mesh=mesh,
      compiler_params=pltpu.CompilerParams(use_tc_tiling_on_sc=False),
      scratch_types=[
          pltpu.VMEM((4, K), jnp.int32),
          pltpu.VMEM((4, K), jnp.int32),
          pltpu.VMEM((REM,), jnp.int32),
          pltpu.VMEM((REM,), jnp.int32),
          pltpu.VMEM((2, K, C), jnp.float32),
          pltpu.VMEM((REM, C), jnp.float32),
          pltpu.VMEM((K, CD), jnp.float32),
          pltpu.VMEM_SHARED((N_ACC, C), jnp.float32),
          pltpu.VMEM_SHARED((N_ACC, CD), jnp.float32),
          pltpu.SemaphoreType.DMA,
          pltpu.SemaphoreType.DMA,
          pltpu.SemaphoreType.DMA,
          pltpu.SemaphoreType.DMA,
      ],
  )
  def body(x_hbm, e_hbm, zf_hbm, zd_hbm, ones_hbm, agg_hbm, deg_hbm,
           row_v, col_v, row_r, col_r, gbuf, gbuf_r, ones_v,
           acc_f, acc_d, sem_i, sem_g, sem_s, sem_d):
    cid = lax.axis_index("c")
    sid = lax.axis_index("s")
    wid = cid * NUM_SUBCORES + sid
    base = pl.multiple_of(wid * EW, 8)

    def eoff(jj):
      return pl.multiple_of(base + jj * K, 8)

    rows = pl.ds(sid * RPT, RPT)
    pltpu.sync_copy(zf_hbm.at[rows], acc_f.at[rows])
    pltpu.sync_copy(zd_hbm.at[rows], acc_d.at[rows])
    pltpu.sync_copy(ones_hbm, ones_v)
    pltpu.async_copy(e_hbm.at[0, pl.ds(eoff(0), K)], row_v.at[0], sem_i)
    pltpu.async_copy(e_hbm.at[1, pl.ds(eoff(0), K)], col_v.at[0], sem_i)
    pltpu.async_copy(e_hbm.at[0, pl.ds(eoff(1), K)], row_v.at[1], sem_i)
    pltpu.async_copy(e_hbm.at[1, pl.ds(eoff(1), K)], col_v.at[1], sem_i)
    plsc.subcore_barrier()
    pltpu.make_async_copy(e_hbm.at[0, pl.ds(eoff(0), K)], row_v.at[0],
                          sem_i).wait()
    pltpu.make_async_copy(e_hbm.at[1, pl.ds(eoff(0), K)], col_v.at[0],
                          sem_i).wait()
    pltpu.async_copy(x_hbm.at[col_v.at[0]], gbuf.at[0], sem_g)

    def step(j, _):
      s2 = lax.rem(j, 2)
      ns2 = lax.rem(j + 1, 2)
      s4 = lax.rem(j, 4)

      @pl.when(j >= 1)
      def _():
        p4 = lax.rem(j + 3, 4)
        pltpu.make_async_copy(gbuf.at[ns2], acc_f.at[row_v.at[p4]],
                              sem_s).wait()
        pltpu.make_async_copy(ones_v, acc_d.at[row_v.at[p4]], sem_d).wait()

      @pl.when(j + 1 < NBF)
      def _():
        n4 = lax.rem(j + 1, 4)
        pltpu.make_async_copy(e_hbm.at[0, pl.ds(eoff(j + 1), K)],
                              row_v.at[n4], sem_i).wait()
        pltpu.make_async_copy(e_hbm.at[1, pl.ds(eoff(j + 1), K)],
                              col_v.at[n4], sem_i).wait()
        pltpu.async_copy(x_hbm.at[col_v.at[n4]], gbuf.at[ns2], sem_g)

      @pl.when(j + 2 < NBF)
      def _():
        n4 = lax.rem(j + 2, 4)
        pltpu.async_copy(e_hbm.at[0, pl.ds(eoff(j + 2), K)],
                         row_v.at[n4], sem_i)
        pltpu.async_copy(e_hbm.at[1, pl.ds(eoff(j + 2), K)],
                         col_v.at[n4], sem_i)

      pltpu.make_async_copy(x_hbm.at[col_v.at[s4]], gbuf.at[s2],
                            sem_g).wait()
      pltpu.async_copy(gbuf.at[s2], acc_f.at[row_v.at[s4]], sem_s,
                       add=True)
      pltpu.async_copy(ones_v, acc_d.at[row_v.at[s4]], sem_d, add=True)
      return 0

    lax.fori_loop(0, NBF, step, 0)
    lp4 = (NBF - 1) % 4
    pltpu.make_async_copy(gbuf.at[(NBF - 1) % 2], acc_f.at[row_v.at[lp4]],
                          sem_s).wait()
    pltpu.make_async_copy(ones_v, acc_d.at[row_v.at[lp4]], sem_d).wait()

    roff = pl.multiple_of(base + NBF * K, 8)
    pltpu.sync_copy(e_hbm.at[0, pl.ds(roff, REM)], row_r)
    pltpu.sync_copy(e_hbm.at[1, pl.ds(roff, REM)], col_r)
    pltpu.async_copy(x_hbm.at[col_r], gbuf_r, sem_g).wait()
    pltpu.async_copy(gbuf_r, acc_f.at[row_r], sem_s, add=True)
    pltpu.async_copy(ones_v.at[pl.ds(0, REM)], acc_d.at[row_r], sem_d,
                     add=True)
    pltpu.make_async_copy(gbuf_r, acc_f.at[row_r], sem_s).wait()
    pltpu.make_async_copy(ones_v.at[pl.ds(0, REM)], acc_d.at[row_r],
                          sem_d).wait()
    plsc.subcore_barrier()

    pltpu.sync_copy(acc_f.at[rows], agg_hbm.at[cid, rows])
    pltpu.sync_copy(acc_d.at[rows], deg_hbm.at[cid, rows])

  return body(node_feats, edge_hbm, zeros_f, zeros_d, ones_kd)


def _combine(agg, deg, w_t, b_row):
  BM = 1000

  def body(a0, a1, d0, d1, wt, bb, o):
    s = a0[0] + a1[0]
    d = d0[0, :, 0:1] + d1[0, :, 0:1]
    o[...] = jnp.dot(s / d, wt[...],
                     preferred_element_type=jnp.float32) + bb[...]

  return pl.pallas_call(
      body,
      grid=(N // BM,),
      in_specs=[
          pl.BlockSpec((1, BM, C), lambda i: (0, i, 0)),
          pl.BlockSpec((1, BM, C), lambda i: (1, i, 0)),
          pl.BlockSpec((1, BM, CD), lambda i: (0, i, 0)),
          pl.BlockSpec((1, BM, CD), lambda i: (1, i, 0)),
          pl.BlockSpec((C, C), lambda i: (0, 0)),
          pl.BlockSpec((1, C), lambda i: (0, 0)),
      ],
      out_specs=pl.BlockSpec((BM, C), lambda i: (i, 0)),
      out_shape=jax.ShapeDtypeStruct((N, C), jnp.float32),
  )(agg, agg, deg, deg, w_t, b_row)


def kernel(node_feats, edge_index, W, b):
  zeros_f = jnp.zeros((N_ACC, C), jnp.float32)
  zeros_d = jnp.zeros((N_ACC, CD), jnp.float32)
  ones_kd = jnp.ones((K, CD), jnp.float32)
  agg, deg = _sc_aggregate(node_feats, edge_index, zeros_f, zeros_d, ones_kd)
  return _combine(agg, deg, W.T, b.reshape(1, C))


# speedup vs baseline: 3.5811x; 1.0715x over previous
import functools

import jax
import jax.numpy as jnp
from jax import lax
from jax.experimental import pallas as pl
from jax.experimental.pallas import tpu as pltpu
from jax.experimental.pallas import tpu_sc as plsc

N = 10000
E = 320000
C = 128
CD = 16
NUM_CORES = 2
NUM_SUBCORES = 16
NW = NUM_CORES * NUM_SUBCORES
EW = E // NW
K = 80
NBF = EW // K
RPT = 632
N_ACC = RPT * NUM_SUBCORES


def _sc_aggregate(node_feats, edge_hbm, zeros_f, zeros_d, ones_kd):
  mesh = plsc.VectorSubcoreMesh(core_axis_name="c", subcore_axis_name="s")

  @functools.partial(
      pl.kernel,
      out_type=(jax.ShapeDtypeStruct((NUM_CORES, N_ACC, C), jnp.float32),
                jax.ShapeDtypeStruct((NUM_CORES, N_ACC, CD), jnp.float32)),
      mesh=mesh,
      compiler_params=pltpu.CompilerParams(use_tc_tiling_on_sc=False),
      scratch_types=[
          pltpu.VMEM((6, K), jnp.int32),
          pltpu.VMEM((6, K), jnp.int32),
          pltpu.VMEM((3, K, C), jnp.float32),
          pltpu.VMEM((K, CD), jnp.float32),
          pltpu.VMEM_SHARED((N_ACC, C), jnp.float32),
          pltpu.VMEM_SHARED((N_ACC, CD), jnp.float32),
          pltpu.SemaphoreType.DMA,
          pltpu.SemaphoreType.DMA,
          pltpu.SemaphoreType.DMA,
          pltpu.SemaphoreType.DMA,
      ],
  )
  def body(x_hbm, e_hbm, zf_hbm, zd_hbm, ones_hbm, agg_hbm, deg_hbm,
           row_v, col_v, gbuf, ones_v,
           acc_f, acc_d, sem_i, sem_g, sem_s, sem_d):
    cid = lax.axis_index("c")
    sid = lax.axis_index("s")
    wid = cid * NUM_SUBCORES + sid
    base = pl.multiple_of(wid * EW, 8)

    def eoff(jj):
      return pl.multiple_of(base + jj * K, 8)

    rows = pl.ds(sid * RPT, RPT)
    pltpu.sync_copy(zf_hbm.at[rows], acc_f.at[rows])
    pltpu.sync_copy(zd_hbm.at[rows], acc_d.at[rows])
    pltpu.sync_copy(ones_hbm, ones_v)
    for jj in range(3):
      pltpu.async_copy(e_hbm.at[0, pl.ds(eoff(jj), K)], row_v.at[jj], sem_i)
      pltpu.async_copy(e_hbm.at[1, pl.ds(eoff(jj), K)], col_v.at[jj], sem_i)
    plsc.subcore_barrier()
    for jj in range(2):
      pltpu.make_async_copy(e_hbm.at[0, pl.ds(eoff(jj), K)], row_v.at[jj],
                            sem_i).wait()
      pltpu.make_async_copy(e_hbm.at[1, pl.ds(eoff(jj), K)], col_v.at[jj],
                            sem_i).wait()
      pltpu.async_copy(x_hbm.at[col_v.at[jj]], gbuf.at[jj], sem_g)

    def step(j, _):
      s3 = lax.rem(j, 3)

      @pl.when(j >= 1)
      def _():
        p6 = lax.rem(j + 5, 6)
        pltpu.make_async_copy(gbuf.at[lax.rem(j + 2, 3)],
                              acc_f.at[row_v.at[p6]], sem_s).wait()
        pltpu.make_async_copy(ones_v, acc_d.at[row_v.at[p6]], sem_d).wait()

      @pl.when(j + 2 < NBF)
      def _():
        n6 = lax.rem(j + 2, 6)
        pltpu.make_async_copy(e_hbm.at[0, pl.ds(eoff(j + 2), K)],
                              row_v.at[n6], sem_i).wait()
        pltpu.make_async_copy(e_hbm.at[1, pl.ds(eoff(j + 2), K)],
                              col_v.at[n6], sem_i).wait()
        pltpu.async_copy(x_hbm.at[col_v.at[n6]], gbuf.at[lax.rem(j + 2, 3)],
                         sem_g)

      @pl.when(j + 3 < NBF)
      def _():
        n6 = lax.rem(j + 3, 6)
        pltpu.async_copy(e_hbm.at[0, pl.ds(eoff(j + 3), K)],
                         row_v.at[n6], sem_i)
        pltpu.async_copy(e_hbm.at[1, pl.ds(eoff(j + 3), K)],
                         col_v.at[n6], sem_i)

      j6 = lax.rem(j, 6)
      pltpu.make_async_copy(x_hbm.at[col_v.at[j6]], gbuf.at[s3],
                            sem_g).wait()
      pltpu.async_copy(gbuf.at[s3], acc_f.at[row_v.at[j6]], sem_s,
                       add=True)
      pltpu.async_copy(ones_v, acc_d.at[row_v.at[j6]], sem_d, add=True)
      return 0

    lax.fori_loop(0, NBF, step, 0)
    lp6 = (NBF - 1) % 6
    pltpu.make_async_copy(gbuf.at[(NBF - 1) % 3], acc_f.at[row_v.at[lp6]],
                          sem_s).wait()
    pltpu.make_async_copy(ones_v, acc_d.at[row_v.at[lp6]], sem_d).wait()
    plsc.subcore_barrier()

    pltpu.sync_copy(acc_f.at[rows], agg_hbm.at[cid, rows])
    pltpu.sync_copy(acc_d.at[rows], deg_hbm.at[cid, rows])

  return body(node_feats, edge_hbm, zeros_f, zeros_d, ones_kd)


def _combine(agg, deg, w_t, b_row):
  BM = 1000

  def body(a0, a1, d0, d1, wt, bb, o):
    s = a0[0] + a1[0]
    d = d0[0, :, 0:1] + d1[0, :, 0:1]
    o[...] = jnp.dot(s / d, wt[...],
                     preferred_element_type=jnp.float32) + bb[...]

  return pl.pallas_call(
      body,
      grid=(N // BM,),
      in_specs=[
          pl.BlockSpec((1, BM, C), lambda i: (0, i, 0)),
          pl.BlockSpec((1, BM, C), lambda i: (1, i, 0)),
          pl.BlockSpec((1, BM, CD), lambda i: (0, i, 0)),
          pl.BlockSpec((1, BM, CD), lambda i: (1, i, 0)),
          pl.BlockSpec((C, C), lambda i: (0, 0)),
          pl.BlockSpec((1, C), lambda i: (0, 0)),
      ],
      out_specs=pl.BlockSpec((BM, C), lambda i: (i, 0)),
      out_shape=jax.ShapeDtypeStruct((N, C), jnp.float32),
  )(agg, agg, deg, deg, w_t, b_row)


def kernel(node_feats, edge_index, W, b):
  zeros_f = jnp.zeros((N_ACC, C), jnp.float32)
  zeros_d = jnp.zeros((N_ACC, CD), jnp.float32)
  ones_kd = jnp.ones((K, CD), jnp.float32)
  agg, deg = _sc_aggregate(node_feats, edge_index, zeros_f, zeros_d, ones_kd)
  return _combine(agg, deg, W.T, b.reshape(1, C))
